# Initial kernel scaffold; baseline (speedup 1.0000x reference)
import jax
import jax.numpy as jnp
from jax.experimental import pallas as pl


def kernel(loc, conf, priors):
    raise NotImplementedError("write your pallas kernel here")


# speedup vs baseline: 22.5658x; 22.5658x over previous
import functools

import jax
import jax.numpy as jnp
from jax import lax
from jax.experimental import pallas as pl
from jax.experimental.pallas import tpu as pltpu
from jax.experimental.pallas import tpu_sc as plsc

NUM_CLASSES = 21
TOP_K = 200
VAR0 = 0.1
VAR1 = 0.2
CONF_THRESH = 0.95
NMS_THRESH = 0.45
BATCH = 4
NUM_PRIORS = 20000

L = 16
NW = 32
NBLK = BATCH * NUM_CLASSES
BLK_PER_W = (NBLK + NW - 1) // NW
NCHUNK_P = NUM_PRIORS // L
TK_PAD = 208
NCHUNK_T = TK_PAD // L
OUT_ROW = TOP_K * 5
OUT_PAD = 1008
NEG_INF = float("-inf")


def _detect_sc(conf_t, loc_t, pri_t):
    mesh = plsc.VectorSubcoreMesh(core_axis_name="c", subcore_axis_name="s",
                                  num_cores=2, num_subcores=16)

    @functools.partial(
        pl.kernel,
        out_type=jax.ShapeDtypeStruct((NBLK, OUT_ROW), jnp.float32),
        mesh=mesh,
        scratch_types=[
            pltpu.VMEM((NUM_PRIORS,), jnp.float32),
            pltpu.VMEM((NUM_PRIORS,), jnp.float32),
            pltpu.VMEM((NUM_PRIORS,), jnp.int32),
            pltpu.VMEM((TK_PAD,), jnp.float32),
            pltpu.VMEM((TK_PAD,), jnp.int32),
            pltpu.VMEM((8 * TK_PAD,), jnp.float32),
            pltpu.VMEM((TK_PAD,), jnp.float32),
            pltpu.VMEM((TK_PAD,), jnp.float32),
            pltpu.VMEM((TK_PAD,), jnp.float32),
            pltpu.VMEM((TK_PAD,), jnp.float32),
            pltpu.VMEM((TK_PAD,), jnp.float32),
            pltpu.VMEM((TK_PAD,), jnp.int32),
            pltpu.VMEM((OUT_PAD,), jnp.float32),
        ],
        compiler_params=pltpu.CompilerParams(use_tc_tiling_on_sc=False,
                                             needs_layout_passes=False),
    )
    def k(conf_hbm, loct_hbm, prit_hbm, out_hbm,
          scores_v, cand_s, cand_i, top_s, top_i, gbuf,
          bx1, by1, bx2, by2, bar, supp, out_buf):
        wid = lax.axis_index("s") * 2 + lax.axis_index("c")
        iota = lax.iota(jnp.int32, L)
        fzero = jnp.zeros((L,), jnp.float32)
        izero = jnp.zeros((L,), jnp.int32)

        def do_block(b, _):
            pb = wid + NW * b

            @pl.when(pb < NBLK)
            def _():
                img = pb // NUM_CLASSES
                cls = pb % NUM_CLASSES

                def zb(c, _c):
                    out_buf[pl.ds(c * L, L)] = fzero
                    return 0
                lax.fori_loop(0, OUT_PAD // L, zb, 0)

                @pl.when(cls > 0)
                def _():
                    pltpu.sync_copy(conf_hbm.at[pb], scores_v)

                    def comp(c, cur):
                        s = scores_v[pl.ds(c * L, L)]
                        m = s > CONF_THRESH
                        pref = plsc.cumsum(m.astype(jnp.int32))
                        dest = cur + pref - 1
                        plsc.store_scatter(cand_s, [dest], s, mask=m)
                        plsc.store_scatter(cand_i, [dest], c * L + iota, mask=m)
                        return cur + plsc.all_reduce_population_count(m)
                    nvec = lax.fori_loop(0, NCHUNK_P, comp, izero)
                    n = jnp.max(nvec)
                    nchunks = (n + L - 1) // L
                    nb = jnp.full((L,), n, jnp.int32)

                    def ext(t, _c):
                        def scan(c, carry):
                            bs, bp = carry
                            s = cand_s[pl.ds(c * L, L)]
                            posv = c * L + iota
                            s = jnp.where(posv < nb, s, NEG_INF)
                            better = s >= bs
                            return (jnp.where(better, s, bs),
                                    jnp.where(better, posv, bp))
                        bs, bp = lax.fori_loop(
                            0, nchunks, scan,
                            (jnp.full((L,), NEG_INF, jnp.float32), izero))
                        smax = jnp.max(bs)
                        smax_v = jnp.full((L,), smax, jnp.float32)
                        pm = jnp.max(jnp.where(bs == smax_v, bp, -1))
                        pm = jnp.maximum(pm, 0)
                        pm_v = jnp.full((L,), pm, jnp.int32)
                        iorig = plsc.load_gather(cand_i, [pm_v])
                        t_v = jnp.full((L,), t, jnp.int32)
                        lane0 = iota == 0
                        plsc.store_scatter(top_s, [t_v], smax_v, mask=lane0)
                        plsc.store_scatter(top_i, [t_v], iorig, mask=lane0)
                        plsc.store_scatter(
                            cand_s, [pm_v],
                            jnp.full((L,), NEG_INF, jnp.float32), mask=lane0)
                        return 0
                    lax.fori_loop(0, TOP_K, ext, 0)

                    for coord in range(4):
                        pltpu.sync_copy(loct_hbm.at[img * 4 + coord], scores_v)

                        def gl(c, _c, coord=coord):
                            ti = top_i[pl.ds(c * L, L)]
                            tic = jnp.clip(ti, 0, NUM_PRIORS - 1)
                            v = plsc.load_gather(scores_v, [tic])
                            gbuf[pl.ds(coord * TK_PAD + c * L, L)] = v
                            return 0
                        lax.fori_loop(0, NCHUNK_T, gl, 0)
                    for coord in range(4):
                        pltpu.sync_copy(prit_hbm.at[coord], scores_v)

                        def gp(c, _c, coord=coord):
                            ti = top_i[pl.ds(c * L, L)]
                            tic = jnp.clip(ti, 0, NUM_PRIORS - 1)
                            v = plsc.load_gather(scores_v, [tic])
                            gbuf[pl.ds((4 + coord) * TK_PAD + c * L, L)] = v
                            return 0
                        lax.fori_loop(0, NCHUNK_T, gp, 0)

                    def dec(c, _c):
                        sl = pl.ds(c * L, L)
                        lx = gbuf[pl.ds(0 * TK_PAD + c * L, L)]
                        ly = gbuf[pl.ds(1 * TK_PAD + c * L, L)]
                        lw = gbuf[pl.ds(2 * TK_PAD + c * L, L)]
                        lh = gbuf[pl.ds(3 * TK_PAD + c * L, L)]
                        px = gbuf[pl.ds(4 * TK_PAD + c * L, L)]
                        py = gbuf[pl.ds(5 * TK_PAD + c * L, L)]
                        pw = gbuf[pl.ds(6 * TK_PAD + c * L, L)]
                        ph = gbuf[pl.ds(7 * TK_PAD + c * L, L)]
                        cx = px + lx * VAR0 * pw
                        cy = py + ly * VAR0 * ph
                        w = pw * jnp.exp(lw * VAR1)
                        h = ph * jnp.exp(lh * VAR1)
                        x1 = cx - w / 2.0
                        y1 = cy - h / 2.0
                        x2 = w + x1
                        y2 = h + y1
                        bx1[sl] = x1
                        by1[sl] = y1
                        bx2[sl] = x2
                        by2[sl] = y2
                        bar[sl] = (x2 - x1) * (y2 - y1)
                        supp[sl] = izero
                        return 0
                    lax.fori_loop(0, NCHUNK_T, dec, 0)

                    def nms(t, cnt):
                        t_v = jnp.full((L,), t, jnp.int32)
                        s_t = plsc.load_gather(top_s, [t_v])
                        sp_t = plsc.load_gather(supp, [t_v])
                        x1t = plsc.load_gather(bx1, [t_v])
                        y1t = plsc.load_gather(by1, [t_v])
                        x2t = plsc.load_gather(bx2, [t_v])
                        y2t = plsc.load_gather(by2, [t_v])
                        art = plsc.load_gather(bar, [t_v])
                        take = (s_t > CONF_THRESH) & (sp_t == 0)

                        vals = jnp.where(
                            iota == 0, s_t,
                            jnp.where(iota == 1, x1t,
                                      jnp.where(iota == 2, y1t,
                                                jnp.where(iota == 3, x2t,
                                                          y2t))))
                        dest = jnp.minimum(cnt * 5 + iota, OUT_PAD - 1)
                        plsc.store_scatter(
                            out_buf, [dest], vals,
                            mask=take & (iota < 5))
                        cnt_new = cnt + jnp.where(take, 1, 0)

                        def sb(c, _c):
                            sl = pl.ds(c * L, L)
                            posv = c * L + iota
                            xx1 = jnp.maximum(bx1[sl], x1t)
                            yy1 = jnp.maximum(by1[sl], y1t)
                            xx2 = jnp.minimum(bx2[sl], x2t)
                            yy2 = jnp.minimum(by2[sl], y2t)
                            w = jnp.maximum(xx2 - xx1, 0.0)
                            h = jnp.maximum(yy2 - yy1, 0.0)
                            inter = w * h
                            union = bar[sl] + art - inter
                            iou = inter / union
                            cond = take & (posv > t_v) & jnp.logical_not(
                                iou <= NMS_THRESH)
                            supp[sl] = jnp.where(cond, 1, supp[sl])
                            return 0
                        lax.fori_loop(t // L, NCHUNK_T, sb, 0)
                        return cnt_new
                    lax.fori_loop(0, TOP_K, nms, izero)

                pltpu.sync_copy(out_buf.at[pl.ds(0, OUT_ROW)], out_hbm.at[pb])
            return 0

        lax.fori_loop(0, BLK_PER_W, do_block, 0)

    return k(conf_t, loc_t, pri_t)


def kernel(loc, conf, priors):
    loc = jax.lax.stop_gradient(loc)
    conf = jax.lax.stop_gradient(conf)
    priors = jax.lax.stop_gradient(priors)
    conf_t = conf.transpose(0, 2, 1).reshape(NBLK, NUM_PRIORS)
    loc_t = loc.transpose(0, 2, 1).reshape(BATCH * 4, NUM_PRIORS)
    pri_t = priors.transpose(1, 0)
    out = _detect_sc(conf_t, loc_t, pri_t)
    return out.reshape(BATCH, NUM_CLASSES, TOP_K, 5)


# speedup vs baseline: 28.4174x; 1.2593x over previous
import functools

import jax
import jax.numpy as jnp
from jax import lax
from jax.experimental import pallas as pl
from jax.experimental.pallas import tpu as pltpu
from jax.experimental.pallas import tpu_sc as plsc

NUM_CLASSES = 21
TOP_K = 200
VAR0 = 0.1
VAR1 = 0.2
CONF_THRESH = 0.95
NMS_THRESH = 0.45
BATCH = 4
NUM_PRIORS = 20000

L = 16
NW = 32
NBLK = BATCH * NUM_CLASSES
BLK_PER_W = (NBLK + NW - 1) // NW
NCHUNK_P = NUM_PRIORS // L
TK_PAD = 208
NCHUNK_T = TK_PAD // L
OUT_ROW = TOP_K * 5
OUT_PAD = 1008
NEG_INF = float("-inf")


def _detect_sc(conf_t, loc_t, pri_t):
    mesh = plsc.VectorSubcoreMesh(core_axis_name="c", subcore_axis_name="s",
                                  num_cores=2, num_subcores=16)

    @functools.partial(
        pl.kernel,
        out_type=jax.ShapeDtypeStruct((NBLK, OUT_ROW), jnp.float32),
        mesh=mesh,
        scratch_types=[
            pltpu.VMEM((NUM_PRIORS,), jnp.float32),
            pltpu.VMEM((NUM_PRIORS,), jnp.float32),
            pltpu.VMEM((NUM_PRIORS,), jnp.int32),
            pltpu.VMEM((NUM_PRIORS,), jnp.int32),
            pltpu.VMEM((TK_PAD,), jnp.float32),
            pltpu.VMEM((TK_PAD,), jnp.int32),
            pltpu.VMEM((8 * TK_PAD,), jnp.float32),
            pltpu.VMEM((TK_PAD,), jnp.float32),
            pltpu.VMEM((TK_PAD,), jnp.float32),
            pltpu.VMEM((TK_PAD,), jnp.float32),
            pltpu.VMEM((TK_PAD,), jnp.float32),
            pltpu.VMEM((TK_PAD,), jnp.float32),
            pltpu.VMEM((TK_PAD,), jnp.int32),
            pltpu.VMEM((OUT_PAD,), jnp.float32),
        ],
        compiler_params=pltpu.CompilerParams(use_tc_tiling_on_sc=False,
                                             needs_layout_passes=False),
    )
    def k(conf_hbm, loct_hbm, prit_hbm, out_hbm,
          scores_v, cand_s, cand_i, cand2_i, top_s, top_i, gbuf,
          bx1, by1, bx2, by2, bar, supp, out_buf):
        wid = lax.axis_index("s") * 2 + lax.axis_index("c")
        iota = lax.iota(jnp.int32, L)
        fzero = jnp.zeros((L,), jnp.float32)
        izero = jnp.zeros((L,), jnp.int32)

        def do_block(b, _):
            pb = wid + NW * b

            @pl.when(pb < NBLK)
            def _():
                img = pb // NUM_CLASSES
                cls = pb % NUM_CLASSES

                def zb(c, _c):
                    out_buf[pl.ds(c * L, L)] = fzero
                    return 0
                lax.fori_loop(0, OUT_PAD // L, zb, 0)

                @pl.when(cls > 0)
                def _():
                    pltpu.sync_copy(conf_hbm.at[pb], scores_v)

                    def comp(c, cur):
                        s = scores_v[pl.ds(c * L, L)]
                        m = s > CONF_THRESH
                        pref = plsc.cumsum(m.astype(jnp.int32))
                        dest = cur + pref - 1
                        plsc.store_scatter(cand_s, [dest], s, mask=m)
                        plsc.store_scatter(cand_i, [dest], c * L + iota, mask=m)
                        return cur + plsc.all_reduce_population_count(m)
                    nvec = lax.fori_loop(0, NCHUNK_P, comp, izero)
                    n = jnp.max(nvec)
                    nchunks = (n + L - 1) // L
                    nb = jnp.full((L,), n, jnp.int32)

                    def count_gt(tv):
                        def cb(c, acc):
                            s = cand_s[pl.ds(c * L, L)]
                            posv = c * L + iota
                            m = (s > tv) & (posv < nb)
                            return acc + plsc.all_reduce_population_count(m)
                        return lax.fori_loop(0, nchunks, cb, izero)

                    def bis(_i, carry):
                        lo, hi = carry
                        mid = (lo + hi) * 0.5
                        cgt = count_gt(mid)
                        big = cgt >= TOP_K
                        return (jnp.where(big, mid, lo),
                                jnp.where(big, hi, mid))
                    lo_v, _hi = lax.fori_loop(
                        0, 24, bis,
                        (jnp.full((L,), CONF_THRESH, jnp.float32),
                         jnp.full((L,), 1.0, jnp.float32)))

                    def comp2(c, cur):
                        s = cand_s[pl.ds(c * L, L)]
                        posv = c * L + iota
                        m = (s > lo_v) & (posv < nb)
                        pref = plsc.cumsum(m.astype(jnp.int32))
                        dest = cur + pref - 1
                        plsc.store_scatter(scores_v, [dest], s, mask=m)
                        ci = cand_i[pl.ds(c * L, L)]
                        plsc.store_scatter(cand2_i, [dest], ci, mask=m)
                        return cur + plsc.all_reduce_population_count(m)
                    n2vec = lax.fori_loop(0, nchunks, comp2, izero)
                    n2 = jnp.max(n2vec)
                    n2chunks = (n2 + L - 1) // L
                    n2b = jnp.full((L,), n2, jnp.int32)

                    def ext(t, _c):
                        def scan(c, carry):
                            bs, bp = carry
                            s = scores_v[pl.ds(c * L, L)]
                            posv = c * L + iota
                            s = jnp.where(posv < n2b, s, NEG_INF)
                            better = s >= bs
                            return (jnp.where(better, s, bs),
                                    jnp.where(better, posv, bp))
                        bs, bp = lax.fori_loop(
                            0, n2chunks, scan,
                            (jnp.full((L,), NEG_INF, jnp.float32), izero))
                        smax = jnp.max(bs)
                        smax_v = jnp.full((L,), smax, jnp.float32)
                        pm = jnp.max(jnp.where(bs == smax_v, bp, -1))
                        pm = jnp.maximum(pm, 0)
                        pm_v = jnp.full((L,), pm, jnp.int32)
                        iorig = plsc.load_gather(cand2_i, [pm_v])
                        t_v = jnp.full((L,), t, jnp.int32)
                        lane0 = iota == 0
                        plsc.store_scatter(top_s, [t_v], smax_v, mask=lane0)
                        plsc.store_scatter(top_i, [t_v], iorig, mask=lane0)
                        plsc.store_scatter(
                            scores_v, [pm_v],
                            jnp.full((L,), NEG_INF, jnp.float32), mask=lane0)
                        return 0
                    lax.fori_loop(0, TOP_K, ext, 0)

                    for coord in range(4):
                        pltpu.sync_copy(loct_hbm.at[img * 4 + coord], scores_v)

                        def gl(c, _c, coord=coord):
                            ti = top_i[pl.ds(c * L, L)]
                            tic = jnp.clip(ti, 0, NUM_PRIORS - 1)
                            v = plsc.load_gather(scores_v, [tic])
                            gbuf[pl.ds(coord * TK_PAD + c * L, L)] = v
                            return 0
                        lax.fori_loop(0, NCHUNK_T, gl, 0)
                    for coord in range(4):
                        pltpu.sync_copy(prit_hbm.at[coord], scores_v)

                        def gp(c, _c, coord=coord):
                            ti = top_i[pl.ds(c * L, L)]
                            tic = jnp.clip(ti, 0, NUM_PRIORS - 1)
                            v = plsc.load_gather(scores_v, [tic])
                            gbuf[pl.ds((4 + coord) * TK_PAD + c * L, L)] = v
                            return 0
                        lax.fori_loop(0, NCHUNK_T, gp, 0)

                    def dec(c, _c):
                        sl = pl.ds(c * L, L)
                        lx = gbuf[pl.ds(0 * TK_PAD + c * L, L)]
                        ly = gbuf[pl.ds(1 * TK_PAD + c * L, L)]
                        lw = gbuf[pl.ds(2 * TK_PAD + c * L, L)]
                        lh = gbuf[pl.ds(3 * TK_PAD + c * L, L)]
                        px = gbuf[pl.ds(4 * TK_PAD + c * L, L)]
                        py = gbuf[pl.ds(5 * TK_PAD + c * L, L)]
                        pw = gbuf[pl.ds(6 * TK_PAD + c * L, L)]
                        ph = gbuf[pl.ds(7 * TK_PAD + c * L, L)]
                        cx = px + lx * VAR0 * pw
                        cy = py + ly * VAR0 * ph
                        w = pw * jnp.exp(lw * VAR1)
                        h = ph * jnp.exp(lh * VAR1)
                        x1 = cx - w / 2.0
                        y1 = cy - h / 2.0
                        x2 = w + x1
                        y2 = h + y1
                        bx1[sl] = x1
                        by1[sl] = y1
                        bx2[sl] = x2
                        by2[sl] = y2
                        bar[sl] = (x2 - x1) * (y2 - y1)
                        supp[sl] = izero
                        return 0
                    lax.fori_loop(0, NCHUNK_T, dec, 0)

                    def nms(t, cnt):
                        t_v = jnp.full((L,), t, jnp.int32)
                        s_t = plsc.load_gather(top_s, [t_v])
                        sp_t = plsc.load_gather(supp, [t_v])
                        x1t = plsc.load_gather(bx1, [t_v])
                        y1t = plsc.load_gather(by1, [t_v])
                        x2t = plsc.load_gather(bx2, [t_v])
                        y2t = plsc.load_gather(by2, [t_v])
                        art = plsc.load_gather(bar, [t_v])
                        take = (s_t > CONF_THRESH) & (sp_t == 0)

                        vals = jnp.where(
                            iota == 0, s_t,
                            jnp.where(iota == 1, x1t,
                                      jnp.where(iota == 2, y1t,
                                                jnp.where(iota == 3, x2t,
                                                          y2t))))
                        dest = jnp.minimum(cnt * 5 + iota, OUT_PAD - 1)
                        plsc.store_scatter(
                            out_buf, [dest], vals,
                            mask=take & (iota < 5))
                        cnt_new = cnt + jnp.where(take, 1, 0)

                        def sb(c, _c):
                            sl = pl.ds(c * L, L)
                            posv = c * L + iota
                            xx1 = jnp.maximum(bx1[sl], x1t)
                            yy1 = jnp.maximum(by1[sl], y1t)
                            xx2 = jnp.minimum(bx2[sl], x2t)
                            yy2 = jnp.minimum(by2[sl], y2t)
                            w = jnp.maximum(xx2 - xx1, 0.0)
                            h = jnp.maximum(yy2 - yy1, 0.0)
                            inter = w * h
                            union = bar[sl] + art - inter
                            iou = inter / union
                            cond = take & (posv > t_v) & jnp.logical_not(
                                iou <= NMS_THRESH)
                            supp[sl] = jnp.where(cond, 1, supp[sl])
                            return 0
                        lax.fori_loop(t // L, NCHUNK_T, sb, 0)
                        return cnt_new
                    lax.fori_loop(0, TOP_K, nms, izero)

                pltpu.sync_copy(out_buf.at[pl.ds(0, OUT_ROW)], out_hbm.at[pb])
            return 0

        lax.fori_loop(0, BLK_PER_W, do_block, 0)

    return k(conf_t, loc_t, pri_t)


def kernel(loc, conf, priors):
    loc = jax.lax.stop_gradient(loc)
    conf = jax.lax.stop_gradient(conf)
    priors = jax.lax.stop_gradient(priors)
    conf_t = conf.transpose(0, 2, 1).reshape(NBLK, NUM_PRIORS)
    loc_t = loc.transpose(0, 2, 1).reshape(BATCH * 4, NUM_PRIORS)
    pri_t = priors.transpose(1, 0)
    out = _detect_sc(conf_t, loc_t, pri_t)
    return out.reshape(BATCH, NUM_CLASSES, TOP_K, 5)


# speedup vs baseline: 29.4951x; 1.0379x over previous
import functools

import jax
import jax.numpy as jnp
from jax import lax
from jax.experimental import pallas as pl
from jax.experimental.pallas import tpu as pltpu
from jax.experimental.pallas import tpu_sc as plsc

NUM_CLASSES = 21
TOP_K = 200
VAR0 = 0.1
VAR1 = 0.2
CONF_THRESH = 0.95
NMS_THRESH = 0.45
BATCH = 4
NUM_PRIORS = 20000

L = 16
NW = 32
NBLK = BATCH * NUM_CLASSES
BLK_PER_W = (NBLK + NW - 1) // NW
NCHUNK_P = NUM_PRIORS // L
TK_PAD = 208
NCHUNK_T = TK_PAD // L
OUT_ROW = TOP_K * 5
OUT_PAD = 1008
NEG_INF = float("-inf")


def _detect_sc(conf_t, loc_t, pri_t):
    mesh = plsc.VectorSubcoreMesh(core_axis_name="c", subcore_axis_name="s",
                                  num_cores=2, num_subcores=16)

    @functools.partial(
        pl.kernel,
        out_type=jax.ShapeDtypeStruct((NBLK, OUT_ROW), jnp.float32),
        mesh=mesh,
        scratch_types=[
            pltpu.VMEM((NUM_PRIORS,), jnp.float32),
            pltpu.VMEM((NUM_PRIORS,), jnp.float32),
            pltpu.VMEM((NUM_PRIORS,), jnp.int32),
            pltpu.VMEM((NUM_PRIORS,), jnp.int32),
            pltpu.VMEM((TK_PAD,), jnp.float32),
            pltpu.VMEM((TK_PAD,), jnp.int32),
            pltpu.VMEM((8 * TK_PAD,), jnp.float32),
            pltpu.VMEM((TK_PAD,), jnp.float32),
            pltpu.VMEM((TK_PAD,), jnp.float32),
            pltpu.VMEM((TK_PAD,), jnp.float32),
            pltpu.VMEM((TK_PAD,), jnp.float32),
            pltpu.VMEM((TK_PAD,), jnp.float32),
            pltpu.VMEM((TK_PAD,), jnp.int32),
            pltpu.VMEM((OUT_PAD,), jnp.float32),
        ],
        compiler_params=pltpu.CompilerParams(use_tc_tiling_on_sc=False,
                                             needs_layout_passes=False),
    )
    def k(conf_hbm, loct_hbm, prit_hbm, out_hbm,
          scores_v, cand_s, cand_i, cand2_i, top_s, top_i, gbuf,
          bx1, by1, bx2, by2, bar, supp, out_buf):
        wid = lax.axis_index("s") * 2 + lax.axis_index("c")
        iota = lax.iota(jnp.int32, L)
        fzero = jnp.zeros((L,), jnp.float32)
        izero = jnp.zeros((L,), jnp.int32)

        def do_block(b, _):
            pb = wid + NW * b

            @pl.when(pb < NBLK)
            def _():
                img = pb // NUM_CLASSES
                cls = pb % NUM_CLASSES

                def zb(c, _c):
                    out_buf[pl.ds(c * L, L)] = fzero
                    return 0
                lax.fori_loop(0, OUT_PAD // L, zb, 0)

                @pl.when(cls > 0)
                def _():
                    pltpu.sync_copy(conf_hbm.at[pb], scores_v)

                    def comp(c, cur):
                        s1 = scores_v[pl.ds(2 * c * L, L)]
                        s2 = scores_v[pl.ds((2 * c + 1) * L, L)]
                        m1 = s1 > CONF_THRESH
                        m2 = s2 > CONF_THRESH
                        c1 = plsc.all_reduce_population_count(m1)
                        c2 = plsc.all_reduce_population_count(m2)
                        d1 = cur + plsc.cumsum(m1.astype(jnp.int32)) - 1
                        d2 = cur + c1 + plsc.cumsum(m2.astype(jnp.int32)) - 1
                        plsc.store_scatter(cand_s, [d1], s1, mask=m1)
                        plsc.store_scatter(cand_i, [d1], 2 * c * L + iota,
                                           mask=m1)
                        plsc.store_scatter(cand_s, [d2], s2, mask=m2)
                        plsc.store_scatter(cand_i, [d2],
                                           (2 * c + 1) * L + iota, mask=m2)
                        return cur + c1 + c2
                    nvec = lax.fori_loop(0, NCHUNK_P // 2, comp, izero)
                    n = jnp.max(nvec)
                    padi = jnp.minimum(nvec + iota, NUM_PRIORS - 1)
                    plsc.store_scatter(
                        cand_s, [padi],
                        jnp.full((L,), NEG_INF, jnp.float32),
                        mask=(nvec + iota) < NUM_PRIORS)
                    nchunks = (n + L - 1) // L
                    nb = jnp.full((L,), n, jnp.int32)

                    def count_gt(tv):
                        def cb(c, acc):
                            s = cand_s[pl.ds(c * L, L)]
                            m = s > tv
                            return acc + plsc.all_reduce_population_count(m)
                        return lax.fori_loop(0, nchunks, cb, izero)

                    def bis(_i, carry):
                        lo, hi = carry
                        mid = (lo + hi) * 0.5
                        cgt = count_gt(mid)
                        big = cgt >= TOP_K
                        return (jnp.where(big, mid, lo),
                                jnp.where(big, hi, mid))
                    lo_v, _hi = lax.fori_loop(
                        0, 24, bis,
                        (jnp.full((L,), CONF_THRESH, jnp.float32),
                         jnp.full((L,), 1.0, jnp.float32)))

                    def comp2(c, cur):
                        s = cand_s[pl.ds(c * L, L)]
                        m = s > lo_v
                        pref = plsc.cumsum(m.astype(jnp.int32))
                        dest = cur + pref - 1
                        plsc.store_scatter(scores_v, [dest], s, mask=m)
                        ci = cand_i[pl.ds(c * L, L)]
                        plsc.store_scatter(cand2_i, [dest], ci, mask=m)
                        return cur + plsc.all_reduce_population_count(m)
                    n2vec = lax.fori_loop(0, nchunks, comp2, izero)
                    n2 = jnp.max(n2vec)
                    n2chunks = (n2 + L - 1) // L
                    padi2 = jnp.minimum(n2vec + iota, NUM_PRIORS - 1)
                    plsc.store_scatter(
                        scores_v, [padi2],
                        jnp.full((L,), NEG_INF, jnp.float32),
                        mask=(n2vec + iota) < NUM_PRIORS)

                    def ext(t, _c):
                        def scan(c, carry):
                            bs, bp = carry
                            s = scores_v[pl.ds(c * L, L)]
                            posv = c * L + iota
                            better = s >= bs
                            return (jnp.where(better, s, bs),
                                    jnp.where(better, posv, bp))
                        bs, bp = lax.fori_loop(
                            0, n2chunks, scan,
                            (jnp.full((L,), NEG_INF, jnp.float32), izero))
                        smax = jnp.max(bs)
                        smax_v = jnp.full((L,), smax, jnp.float32)
                        pm = jnp.max(jnp.where(bs == smax_v, bp, -1))
                        pm = jnp.maximum(pm, 0)
                        pm_v = jnp.full((L,), pm, jnp.int32)
                        iorig = plsc.load_gather(cand2_i, [pm_v])
                        t_v = jnp.full((L,), t, jnp.int32)
                        lane0 = iota == 0
                        plsc.store_scatter(top_s, [t_v], smax_v, mask=lane0)
                        plsc.store_scatter(top_i, [t_v], iorig, mask=lane0)
                        plsc.store_scatter(
                            scores_v, [pm_v],
                            jnp.full((L,), NEG_INF, jnp.float32), mask=lane0)
                        return 0
                    lax.fori_loop(0, TOP_K, ext, 0)

                    for coord in range(4):
                        pltpu.sync_copy(loct_hbm.at[img * 4 + coord], scores_v)

                        def gl(c, _c, coord=coord):
                            ti = top_i[pl.ds(c * L, L)]
                            tic = jnp.clip(ti, 0, NUM_PRIORS - 1)
                            v = plsc.load_gather(scores_v, [tic])
                            gbuf[pl.ds(coord * TK_PAD + c * L, L)] = v
                            return 0
                        lax.fori_loop(0, NCHUNK_T, gl, 0)
                    for coord in range(4):
                        pltpu.sync_copy(prit_hbm.at[coord], scores_v)

                        def gp(c, _c, coord=coord):
                            ti = top_i[pl.ds(c * L, L)]
                            tic = jnp.clip(ti, 0, NUM_PRIORS - 1)
                            v = plsc.load_gather(scores_v, [tic])
                            gbuf[pl.ds((4 + coord) * TK_PAD + c * L, L)] = v
                            return 0
                        lax.fori_loop(0, NCHUNK_T, gp, 0)

                    def dec(c, _c):
                        sl = pl.ds(c * L, L)
                        lx = gbuf[pl.ds(0 * TK_PAD + c * L, L)]
                        ly = gbuf[pl.ds(1 * TK_PAD + c * L, L)]
                        lw = gbuf[pl.ds(2 * TK_PAD + c * L, L)]
                        lh = gbuf[pl.ds(3 * TK_PAD + c * L, L)]
                        px = gbuf[pl.ds(4 * TK_PAD + c * L, L)]
                        py = gbuf[pl.ds(5 * TK_PAD + c * L, L)]
                        pw = gbuf[pl.ds(6 * TK_PAD + c * L, L)]
                        ph = gbuf[pl.ds(7 * TK_PAD + c * L, L)]
                        cx = px + lx * VAR0 * pw
                        cy = py + ly * VAR0 * ph
                        w = pw * jnp.exp(lw * VAR1)
                        h = ph * jnp.exp(lh * VAR1)
                        x1 = cx - w / 2.0
                        y1 = cy - h / 2.0
                        x2 = w + x1
                        y2 = h + y1
                        bx1[sl] = x1
                        by1[sl] = y1
                        bx2[sl] = x2
                        by2[sl] = y2
                        bar[sl] = (x2 - x1) * (y2 - y1)
                        supp[sl] = izero
                        return 0
                    lax.fori_loop(0, NCHUNK_T, dec, 0)

                    def nms(t, cnt):
                        t_v = jnp.full((L,), t, jnp.int32)
                        s_t = plsc.load_gather(top_s, [t_v])
                        sp_t = plsc.load_gather(supp, [t_v])
                        x1t = plsc.load_gather(bx1, [t_v])
                        y1t = plsc.load_gather(by1, [t_v])
                        x2t = plsc.load_gather(bx2, [t_v])
                        y2t = plsc.load_gather(by2, [t_v])
                        art = plsc.load_gather(bar, [t_v])
                        take = (s_t > CONF_THRESH) & (sp_t == 0)

                        vals = jnp.where(
                            iota == 0, s_t,
                            jnp.where(iota == 1, x1t,
                                      jnp.where(iota == 2, y1t,
                                                jnp.where(iota == 3, x2t,
                                                          y2t))))
                        dest = jnp.minimum(cnt * 5 + iota, OUT_PAD - 1)
                        plsc.store_scatter(
                            out_buf, [dest], vals,
                            mask=take & (iota < 5))
                        cnt_new = cnt + jnp.where(take, 1, 0)

                        def sb(c, _c):
                            sl = pl.ds(c * L, L)
                            posv = c * L + iota
                            xx1 = jnp.maximum(bx1[sl], x1t)
                            yy1 = jnp.maximum(by1[sl], y1t)
                            xx2 = jnp.minimum(bx2[sl], x2t)
                            yy2 = jnp.minimum(by2[sl], y2t)
                            w = jnp.maximum(xx2 - xx1, 0.0)
                            h = jnp.maximum(yy2 - yy1, 0.0)
                            inter = w * h
                            union = bar[sl] + art - inter
                            iou = inter / union
                            cond = take & (posv > t_v) & jnp.logical_not(
                                iou <= NMS_THRESH)
                            supp[sl] = jnp.where(cond, 1, supp[sl])
                            return 0
                        lax.fori_loop(t // L, NCHUNK_T, sb, 0)
                        return cnt_new
                    lax.fori_loop(0, TOP_K, nms, izero)

                pltpu.sync_copy(out_buf.at[pl.ds(0, OUT_ROW)], out_hbm.at[pb])
            return 0

        lax.fori_loop(0, BLK_PER_W, do_block, 0)

    return k(conf_t, loc_t, pri_t)


def kernel(loc, conf, priors):
    loc = jax.lax.stop_gradient(loc)
    conf = jax.lax.stop_gradient(conf)
    priors = jax.lax.stop_gradient(priors)
    conf_t = conf.transpose(0, 2, 1).reshape(NBLK, NUM_PRIORS)
    loc_t = loc.transpose(0, 2, 1).reshape(BATCH * 4, NUM_PRIORS)
    pri_t = priors.transpose(1, 0)
    out = _detect_sc(conf_t, loc_t, pri_t)
    return out.reshape(BATCH, NUM_CLASSES, TOP_K, 5)


# speedup vs baseline: 30.4055x; 1.0309x over previous
import functools

import jax
import jax.numpy as jnp
from jax import lax
from jax.experimental import pallas as pl
from jax.experimental.pallas import tpu as pltpu
from jax.experimental.pallas import tpu_sc as plsc

NUM_CLASSES = 21
TOP_K = 200
VAR0 = 0.1
VAR1 = 0.2
CONF_THRESH = 0.95
NMS_THRESH = 0.45
BATCH = 4
NUM_PRIORS = 20000

L = 16
NW = 32
NBLK = BATCH * NUM_CLASSES
BLK_PER_W = (NBLK + NW - 1) // NW
NCHUNK_P = NUM_PRIORS // L
TK_PAD = 208
NCHUNK_T = TK_PAD // L
OUT_ROW = TOP_K * 5
OUT_PAD = 1008
NEG_INF = float("-inf")


def _detect_sc(conf_t, loc_t, pri_t):
    mesh = plsc.VectorSubcoreMesh(core_axis_name="c", subcore_axis_name="s",
                                  num_cores=2, num_subcores=16)

    @functools.partial(
        pl.kernel,
        out_type=jax.ShapeDtypeStruct((NBLK, OUT_ROW), jnp.float32),
        mesh=mesh,
        scratch_types=[
            pltpu.VMEM((NUM_PRIORS,), jnp.float32),
            pltpu.VMEM((NUM_PRIORS,), jnp.float32),
            pltpu.VMEM((NUM_PRIORS,), jnp.int32),
            pltpu.VMEM((NUM_PRIORS,), jnp.int32),
            pltpu.VMEM((TK_PAD,), jnp.float32),
            pltpu.VMEM((TK_PAD,), jnp.int32),
            pltpu.VMEM((8 * TK_PAD,), jnp.float32),
            pltpu.VMEM((TK_PAD,), jnp.float32),
            pltpu.VMEM((TK_PAD,), jnp.float32),
            pltpu.VMEM((TK_PAD,), jnp.float32),
            pltpu.VMEM((TK_PAD,), jnp.float32),
            pltpu.VMEM((TK_PAD,), jnp.float32),
            pltpu.VMEM((TK_PAD,), jnp.int32),
            pltpu.VMEM((OUT_PAD,), jnp.float32),
            pltpu.VMEM((NUM_PRIORS,), jnp.float32),
            pltpu.SemaphoreType.DMA,
            pltpu.SemaphoreType.DMA,
        ],
        compiler_params=pltpu.CompilerParams(use_tc_tiling_on_sc=False,
                                             needs_layout_passes=False),
    )
    def k(conf_hbm, loct_hbm, prit_hbm, out_hbm,
          scores_v, cand_s, cand_i, cand2_i, top_s, top_i, gbuf,
          bx1, by1, bx2, by2, bar, supp, out_buf, stage_b, sem_a, sem_b):
        wid = lax.axis_index("s") * 2 + lax.axis_index("c")
        iota = lax.iota(jnp.int32, L)
        fzero = jnp.zeros((L,), jnp.float32)
        izero = jnp.zeros((L,), jnp.int32)

        def do_block(b, _):
            pb = wid + NW * b

            @pl.when(pb < NBLK)
            def _():
                img = pb // NUM_CLASSES
                cls = pb % NUM_CLASSES

                def zb(c, _c):
                    out_buf[pl.ds(c * L, L)] = fzero
                    return 0
                lax.fori_loop(0, OUT_PAD // L, zb, 0)

                @pl.when(cls > 0)
                def _():
                    pltpu.sync_copy(conf_hbm.at[pb], scores_v)

                    def comp(c, cur):
                        s1 = scores_v[pl.ds(2 * c * L, L)]
                        s2 = scores_v[pl.ds((2 * c + 1) * L, L)]
                        m1 = s1 > CONF_THRESH
                        m2 = s2 > CONF_THRESH
                        c1 = plsc.all_reduce_population_count(m1)
                        c2 = plsc.all_reduce_population_count(m2)
                        d1 = cur + plsc.cumsum(m1.astype(jnp.int32)) - 1
                        d2 = cur + c1 + plsc.cumsum(m2.astype(jnp.int32)) - 1
                        plsc.store_scatter(cand_s, [d1], s1, mask=m1)
                        plsc.store_scatter(cand_i, [d1], 2 * c * L + iota,
                                           mask=m1)
                        plsc.store_scatter(cand_s, [d2], s2, mask=m2)
                        plsc.store_scatter(cand_i, [d2],
                                           (2 * c + 1) * L + iota, mask=m2)
                        return cur + c1 + c2
                    nvec = lax.fori_loop(0, NCHUNK_P // 2, comp, izero)
                    n = jnp.max(nvec)
                    padi = jnp.minimum(nvec + iota, NUM_PRIORS - 1)
                    plsc.store_scatter(
                        cand_s, [padi],
                        jnp.full((L,), NEG_INF, jnp.float32),
                        mask=(nvec + iota) < NUM_PRIORS)
                    nchunks = (n + L - 1) // L
                    nb = jnp.full((L,), n, jnp.int32)

                    def count_gt(tv):
                        def cb(c, acc):
                            s = cand_s[pl.ds(c * L, L)]
                            m = s > tv
                            return acc + plsc.all_reduce_population_count(m)
                        return lax.fori_loop(0, nchunks, cb, izero)

                    def bis(_i, carry):
                        lo, hi = carry
                        mid = (lo + hi) * 0.5
                        cgt = count_gt(mid)
                        big = cgt >= TOP_K
                        return (jnp.where(big, mid, lo),
                                jnp.where(big, hi, mid))
                    lo_v, _hi = lax.fori_loop(
                        0, 24, bis,
                        (jnp.full((L,), CONF_THRESH, jnp.float32),
                         jnp.full((L,), 1.0, jnp.float32)))

                    def comp2(c, cur):
                        s = cand_s[pl.ds(c * L, L)]
                        m = s > lo_v
                        pref = plsc.cumsum(m.astype(jnp.int32))
                        dest = cur + pref - 1
                        plsc.store_scatter(scores_v, [dest], s, mask=m)
                        ci = cand_i[pl.ds(c * L, L)]
                        plsc.store_scatter(cand2_i, [dest], ci, mask=m)
                        return cur + plsc.all_reduce_population_count(m)
                    n2vec = lax.fori_loop(0, nchunks, comp2, izero)
                    n2 = jnp.max(n2vec)
                    n2chunks = (n2 + L - 1) // L
                    padi2 = jnp.minimum(n2vec + iota, NUM_PRIORS - 1)
                    plsc.store_scatter(
                        scores_v, [padi2],
                        jnp.full((L,), NEG_INF, jnp.float32),
                        mask=(n2vec + iota) < NUM_PRIORS)

                    def ext(t, _c):
                        def scan(c, carry):
                            bs, bp = carry
                            s = scores_v[pl.ds(c * L, L)]
                            posv = c * L + iota
                            better = s >= bs
                            return (jnp.where(better, s, bs),
                                    jnp.where(better, posv, bp))
                        bs, bp = lax.fori_loop(
                            0, n2chunks, scan,
                            (jnp.full((L,), NEG_INF, jnp.float32), izero))
                        smax = jnp.max(bs)
                        smax_v = jnp.full((L,), smax, jnp.float32)
                        pm = jnp.max(jnp.where(bs == smax_v, bp, -1))
                        pm = jnp.maximum(pm, 0)
                        pm_v = jnp.full((L,), pm, jnp.int32)
                        iorig = plsc.load_gather(cand2_i, [pm_v])
                        t_v = jnp.full((L,), t, jnp.int32)
                        lane0 = iota == 0
                        plsc.store_scatter(top_s, [t_v], smax_v, mask=lane0)
                        plsc.store_scatter(top_i, [t_v], iorig, mask=lane0)
                        plsc.store_scatter(
                            scores_v, [pm_v],
                            jnp.full((L,), NEG_INF, jnp.float32), mask=lane0)
                        return 0
                    lax.fori_loop(0, TOP_K, ext, 0)

                    bufs = (scores_v, stage_b)
                    sems = (sem_a, sem_b)

                    def col_src(j):
                        if j < 4:
                            return loct_hbm.at[img * 4 + j]
                        return prit_hbm.at[j - 4]

                    cps = {0: pltpu.async_copy(col_src(0), bufs[0], sems[0])}
                    for j in range(8):
                        if j < 7:
                            cps[j + 1] = pltpu.async_copy(
                                col_src(j + 1), bufs[(j + 1) % 2],
                                sems[(j + 1) % 2])
                        cps[j].wait()
                        buf = bufs[j % 2]

                        def gl(c, _c, j=j, buf=buf):
                            ti = top_i[pl.ds(c * L, L)]
                            tic = jnp.clip(ti, 0, NUM_PRIORS - 1)
                            v = plsc.load_gather(buf, [tic])
                            gbuf[pl.ds(j * TK_PAD + c * L, L)] = v
                            return 0
                        lax.fori_loop(0, NCHUNK_T, gl, 0)

                    def dec(c, _c):
                        sl = pl.ds(c * L, L)
                        lx = gbuf[pl.ds(0 * TK_PAD + c * L, L)]
                        ly = gbuf[pl.ds(1 * TK_PAD + c * L, L)]
                        lw = gbuf[pl.ds(2 * TK_PAD + c * L, L)]
                        lh = gbuf[pl.ds(3 * TK_PAD + c * L, L)]
                        px = gbuf[pl.ds(4 * TK_PAD + c * L, L)]
                        py = gbuf[pl.ds(5 * TK_PAD + c * L, L)]
                        pw = gbuf[pl.ds(6 * TK_PAD + c * L, L)]
                        ph = gbuf[pl.ds(7 * TK_PAD + c * L, L)]
                        cx = px + lx * VAR0 * pw
                        cy = py + ly * VAR0 * ph
                        w = pw * jnp.exp(lw * VAR1)
                        h = ph * jnp.exp(lh * VAR1)
                        x1 = cx - w / 2.0
                        y1 = cy - h / 2.0
                        x2 = w + x1
                        y2 = h + y1
                        bx1[sl] = x1
                        by1[sl] = y1
                        bx2[sl] = x2
                        by2[sl] = y2
                        bar[sl] = (x2 - x1) * (y2 - y1)
                        supp[sl] = izero
                        return 0
                    lax.fori_loop(0, NCHUNK_T, dec, 0)

                    def nms(t, cnt):
                        t_v = jnp.full((L,), t, jnp.int32)
                        s_t = plsc.load_gather(top_s, [t_v])
                        sp_t = plsc.load_gather(supp, [t_v])
                        x1t = plsc.load_gather(bx1, [t_v])
                        y1t = plsc.load_gather(by1, [t_v])
                        x2t = plsc.load_gather(bx2, [t_v])
                        y2t = plsc.load_gather(by2, [t_v])
                        art = plsc.load_gather(bar, [t_v])
                        take = (s_t > CONF_THRESH) & (sp_t == 0)

                        vals = jnp.where(
                            iota == 0, s_t,
                            jnp.where(iota == 1, x1t,
                                      jnp.where(iota == 2, y1t,
                                                jnp.where(iota == 3, x2t,
                                                          y2t))))
                        dest = jnp.minimum(cnt * 5 + iota, OUT_PAD - 1)
                        plsc.store_scatter(
                            out_buf, [dest], vals,
                            mask=take & (iota < 5))
                        cnt_new = cnt + jnp.where(take, 1, 0)

                        def sb(c, _c):
                            sl = pl.ds(c * L, L)
                            posv = c * L + iota
                            xx1 = jnp.maximum(bx1[sl], x1t)
                            yy1 = jnp.maximum(by1[sl], y1t)
                            xx2 = jnp.minimum(bx2[sl], x2t)
                            yy2 = jnp.minimum(by2[sl], y2t)
                            w = jnp.maximum(xx2 - xx1, 0.0)
                            h = jnp.maximum(yy2 - yy1, 0.0)
                            inter = w * h
                            union = bar[sl] + art - inter
                            iou = inter / union
                            cond = take & (posv > t_v) & jnp.logical_not(
                                iou <= NMS_THRESH)
                            supp[sl] = jnp.where(cond, 1, supp[sl])
                            return 0
                        lax.fori_loop(t // L, NCHUNK_T, sb, 0)
                        return cnt_new
                    lax.fori_loop(0, TOP_K, nms, izero)

                pltpu.sync_copy(out_buf.at[pl.ds(0, OUT_ROW)], out_hbm.at[pb])
            return 0

        lax.fori_loop(0, BLK_PER_W, do_block, 0)

    return k(conf_t, loc_t, pri_t)


def kernel(loc, conf, priors):
    loc = jax.lax.stop_gradient(loc)
    conf = jax.lax.stop_gradient(conf)
    priors = jax.lax.stop_gradient(priors)
    conf_t = conf.transpose(0, 2, 1).reshape(NBLK, NUM_PRIORS)
    loc_t = loc.transpose(0, 2, 1).reshape(BATCH * 4, NUM_PRIORS)
    pri_t = priors.transpose(1, 0)
    out = _detect_sc(conf_t, loc_t, pri_t)
    return out.reshape(BATCH, NUM_CLASSES, TOP_K, 5)
